# Initial kernel scaffold; baseline (speedup 1.0000x reference)
import functools

import jax
import jax.numpy as jnp
from jax import lax
from jax.experimental import pallas as pl
from jax.experimental.pallas import tpu as pltpu
from jax.experimental.pallas import tpu_sc as plsc

N = 10000
E = 320000
D = 128
NCORES = 2
NSUB = 16
NW = NCORES * NSUB
EPT = E // NW
CH = 80
NCH = EPT // CH
PAD_N = 10240
STRIPE = PAD_N // NSUB
RB = 125
RPT = N // NSUB

_MESH = plsc.VectorSubcoreMesh(core_axis_name="c", subcore_axis_name="s")


def _sc_degree(row):
    @functools.partial(
        pl.kernel,
        out_type=jax.ShapeDtypeStruct((NCORES, PAD_N), jnp.float32),
        mesh=_MESH,
        scratch_types=[
            pltpu.VMEM((CH,), jnp.int32),
            pltpu.VMEM((CH,), jnp.float32),
            pltpu.VMEM((STRIPE,), jnp.float32),
        ],
    )
    def k(row_hbm, out_hbm, idx_v, ones_v, buf_v):
        def inner(deg_sh):
            c = lax.axis_index("c")
            s = lax.axis_index("s")
            base = (c * NSUB + s) * EPT
            one16 = jnp.ones((16,), jnp.float32)
            zero16 = jnp.zeros((16,), jnp.float32)
            for i in range(CH // 16):
                ones_v[pl.ds(i * 16, 16)] = one16

            def zb(i, _):
                buf_v[pl.ds(i * 16, 16)] = zero16
                return 0

            lax.fori_loop(0, STRIPE // 16, zb, 0)
            pltpu.sync_copy(buf_v, deg_sh.at[pl.ds(s * STRIPE, STRIPE)])
            plsc.subcore_barrier()

            def body(j, _):
                pltpu.sync_copy(row_hbm.at[pl.ds(base + j * CH, CH)], idx_v)
                pltpu.sync_copy(ones_v, deg_sh.at[idx_v], add=True)
                return 0

            lax.fori_loop(0, NCH, body, 0)
            plsc.subcore_barrier()
            pltpu.sync_copy(deg_sh.at[pl.ds(s * STRIPE, STRIPE)], buf_v)
            pltpu.sync_copy(buf_v, out_hbm.at[c, pl.ds(s * STRIPE, STRIPE)])

        pl.run_scoped(inner, pltpu.VMEM_SHARED((PAD_N,), jnp.float32))

    return k(row)


def _sc_aggregate(col, row, hp):
    @functools.partial(
        pl.kernel,
        out_type=jax.ShapeDtypeStruct((NCORES, N, D), jnp.float32),
        mesh=_MESH,
        scratch_types=[
            pltpu.VMEM((CH,), jnp.int32),
            pltpu.VMEM((CH,), jnp.int32),
            pltpu.VMEM((CH, D), jnp.float32),
            pltpu.VMEM((RB, D), jnp.float32),
            pltpu.SemaphoreType.DMA,
        ],
    )
    def k(col_hbm, row_hbm, hp_hbm, out_hbm, ci_v, ri_v, rows_v, buf_v, sem):
        def inner(agg_sh):
            c = lax.axis_index("c")
            s = lax.axis_index("s")
            base = (c * NSUB + s) * EPT
            zero16 = jnp.zeros((16,), jnp.float32)

            def zb(i, _):
                r = i // (D // 16)
                q = i % (D // 16)
                buf_v[r, pl.ds(q * 16, 16)] = zero16
                return 0

            lax.fori_loop(0, RB * (D // 16), zb, 0)
            for t in range(RPT // RB):
                pltpu.sync_copy(buf_v, agg_sh.at[pl.ds(s * RPT + t * RB, RB)])
            plsc.subcore_barrier()

            def body(j, _):
                e0 = base + j * CH
                pltpu.sync_copy(col_hbm.at[pl.ds(e0, CH)], ci_v)
                pltpu.sync_copy(row_hbm.at[pl.ds(e0, CH)], ri_v)
                pltpu.async_copy(hp_hbm.at[ci_v], rows_v, sem).wait()
                pltpu.sync_copy(rows_v, agg_sh.at[ri_v], add=True)
                return 0

            lax.fori_loop(0, NCH, body, 0)
            plsc.subcore_barrier()
            for t in range(RPT // RB):
                pltpu.sync_copy(agg_sh.at[pl.ds(s * RPT + t * RB, RB)], buf_v)
                pltpu.sync_copy(buf_v, out_hbm.at[c, pl.ds(s * RPT + t * RB, RB)])

        pl.run_scoped(inner, pltpu.VMEM_SHARED((N, D), jnp.float32))

    return k(col, row, hp)


_M_BLK = 1000


def _tc_hprime(x, w, deg_col):
    def body(x_ref, w_ref, d_ref, o_ref):
        dis = lax.rsqrt(d_ref[...])
        h = jnp.dot(x_ref[...], w_ref[...], preferred_element_type=jnp.float32)
        o_ref[...] = dis * h

    return pl.pallas_call(
        body,
        grid=(N // _M_BLK,),
        in_specs=[
            pl.BlockSpec((_M_BLK, D), lambda i: (i, 0)),
            pl.BlockSpec((D, D), lambda i: (0, 0)),
            pl.BlockSpec((_M_BLK, 1), lambda i: (i, 0)),
        ],
        out_specs=pl.BlockSpec((_M_BLK, D), lambda i: (i, 0)),
        out_shape=jax.ShapeDtypeStruct((N, D), jnp.float32),
    )(x, w, deg_col)


def _tc_final(aggp, hp, deg_col, bias2d):
    def body(a_ref, h_ref, d_ref, b_ref, o_ref):
        dis = lax.rsqrt(d_ref[...])
        o_ref[...] = dis * (a_ref[0] + a_ref[1] + h_ref[...]) + b_ref[...]

    return pl.pallas_call(
        body,
        grid=(N // _M_BLK,),
        in_specs=[
            pl.BlockSpec((NCORES, _M_BLK, D), lambda i: (0, i, 0)),
            pl.BlockSpec((_M_BLK, D), lambda i: (i, 0)),
            pl.BlockSpec((_M_BLK, 1), lambda i: (i, 0)),
            pl.BlockSpec((1, D), lambda i: (0, 0)),
        ],
        out_specs=pl.BlockSpec((_M_BLK, D), lambda i: (i, 0)),
        out_shape=jax.ShapeDtypeStruct((N, D), jnp.float32),
    )(aggp, hp, deg_col, bias2d)


def kernel(x, edge_index, kernel, bias):
    row = edge_index[0]
    col = edge_index[1]
    degp = _sc_degree(row)
    deg_col = (degp[0, :N] + degp[1, :N] + 1.0)[:, None]
    hp = _tc_hprime(x, kernel, deg_col)
    aggp = _sc_aggregate(col, row, hp)
    return _tc_final(aggp, hp, deg_col, bias[None, :])


# speedup vs baseline: 17.7677x; 17.7677x over previous
import functools

import jax
import jax.numpy as jnp
from jax import lax
from jax.experimental import pallas as pl
from jax.experimental.pallas import tpu as pltpu
from jax.experimental.pallas import tpu_sc as plsc

N = 10000
E = 320000
D = 128
NCORES = 2
NSUB = 16
NW = NCORES * NSUB
EPT = E // NW
CH = 80
NCH = EPT // CH
PAD_N = 10240
STRIPE = PAD_N // NSUB
RB = 128

_MESH = plsc.VectorSubcoreMesh(core_axis_name="c", subcore_axis_name="s")


def _sc_degree(row):
    @functools.partial(
        pl.kernel,
        out_type=jax.ShapeDtypeStruct((NCORES, PAD_N), jnp.float32),
        mesh=_MESH,
        scratch_types=[
            pltpu.VMEM((CH,), jnp.int32),
            pltpu.VMEM((CH,), jnp.float32),
            pltpu.VMEM((STRIPE,), jnp.float32),
            pltpu.VMEM_SHARED((PAD_N,), jnp.float32),
        ],
    )
    def k(row_hbm, out_hbm, idx_v, ones_v, buf_v, deg_sh):
        c = lax.axis_index("c")
        s = lax.axis_index("s")
        base = (c * NSUB + s) * EPT
        one16 = jnp.ones((16,), jnp.float32)
        zero16 = jnp.zeros((16,), jnp.float32)
        for i in range(CH // 16):
            ones_v[pl.ds(i * 16, 16)] = one16

        def zb(i, _):
            buf_v[pl.ds(i * 16, 16)] = zero16
            return 0

        lax.fori_loop(0, STRIPE // 16, zb, 0)
        pltpu.sync_copy(buf_v, deg_sh.at[pl.ds(s * STRIPE, STRIPE)])
        plsc.subcore_barrier()

        def body(j, _):
            pltpu.sync_copy(row_hbm.at[pl.ds(base + j * CH, CH)], idx_v)
            pltpu.sync_copy(ones_v, deg_sh.at[idx_v], add=True)
            return 0

        lax.fori_loop(0, NCH, body, 0)
        plsc.subcore_barrier()
        pltpu.sync_copy(deg_sh.at[pl.ds(s * STRIPE, STRIPE)], buf_v)
        pltpu.sync_copy(buf_v, out_hbm.at[c, pl.ds(s * STRIPE, STRIPE)])

    return k(row)


def _sc_aggregate(col, row, hp):
    @functools.partial(
        pl.kernel,
        out_type=jax.ShapeDtypeStruct((NCORES, PAD_N, D), jnp.float32),
        mesh=_MESH,
        scratch_types=[
            pltpu.VMEM((CH,), jnp.int32),
            pltpu.VMEM((CH,), jnp.int32),
            pltpu.VMEM((CH, D), jnp.float32),
            pltpu.VMEM((RB, D), jnp.float32),
            pltpu.VMEM_SHARED((PAD_N, D), jnp.float32),
            pltpu.SemaphoreType.DMA,
        ],
    )
    def k(col_hbm, row_hbm, hp_hbm, out_hbm, ci_v, ri_v, rows_v, buf_v, agg_sh, sem):
        c = lax.axis_index("c")
        s = lax.axis_index("s")
        base = (c * NSUB + s) * EPT
        zero16 = jnp.zeros((16,), jnp.float32)

        def zb(i, _):
            r = i // (D // 16)
            q = i % (D // 16)
            buf_v[r, pl.ds(q * 16, 16)] = zero16
            return 0

        lax.fori_loop(0, RB * (D // 16), zb, 0)
        for t in range(STRIPE // RB):
            pltpu.sync_copy(buf_v, agg_sh.at[pl.ds(s * STRIPE + t * RB, RB)])
        plsc.subcore_barrier()

        def body(j, _):
            e0 = base + j * CH
            pltpu.sync_copy(col_hbm.at[pl.ds(e0, CH)], ci_v)
            pltpu.sync_copy(row_hbm.at[pl.ds(e0, CH)], ri_v)
            pltpu.async_copy(hp_hbm.at[ci_v], rows_v, sem).wait()
            pltpu.sync_copy(rows_v, agg_sh.at[ri_v], add=True)
            return 0

        lax.fori_loop(0, NCH, body, 0)
        plsc.subcore_barrier()
        for t in range(STRIPE // RB):
            pltpu.sync_copy(agg_sh.at[pl.ds(s * STRIPE + t * RB, RB)], buf_v)
            pltpu.sync_copy(buf_v, out_hbm.at[c, pl.ds(s * STRIPE + t * RB, RB)])

    return k(col, row, hp)


_M_BLK = 1000


def _tc_hprime(x, w, deg_col):
    def body(x_ref, w_ref, d_ref, o_ref):
        dis = lax.rsqrt(d_ref[...])
        h = jnp.dot(x_ref[...], w_ref[...], preferred_element_type=jnp.float32)
        o_ref[...] = dis * h

    return pl.pallas_call(
        body,
        grid=(N // _M_BLK,),
        in_specs=[
            pl.BlockSpec((_M_BLK, D), lambda i: (i, 0)),
            pl.BlockSpec((D, D), lambda i: (0, 0)),
            pl.BlockSpec((_M_BLK, 1), lambda i: (i, 0)),
        ],
        out_specs=pl.BlockSpec((_M_BLK, D), lambda i: (i, 0)),
        out_shape=jax.ShapeDtypeStruct((N, D), jnp.float32),
    )(x, w, deg_col)


def _tc_final(aggp, hp, deg_col, bias2d):
    def body(a_ref, h_ref, d_ref, b_ref, o_ref):
        dis = lax.rsqrt(d_ref[...])
        o_ref[...] = dis * (a_ref[0] + a_ref[1] + h_ref[...]) + b_ref[...]

    return pl.pallas_call(
        body,
        grid=(N // _M_BLK,),
        in_specs=[
            pl.BlockSpec((NCORES, _M_BLK, D), lambda i: (0, i, 0)),
            pl.BlockSpec((_M_BLK, D), lambda i: (i, 0)),
            pl.BlockSpec((_M_BLK, 1), lambda i: (i, 0)),
            pl.BlockSpec((1, D), lambda i: (0, 0)),
        ],
        out_specs=pl.BlockSpec((_M_BLK, D), lambda i: (i, 0)),
        out_shape=jax.ShapeDtypeStruct((N, D), jnp.float32),
    )(aggp, hp, deg_col, bias2d)


def kernel(x, edge_index, kernel, bias):
    row = edge_index[0]
    col = edge_index[1]
    degp = _sc_degree(row)
    deg_col = (degp[0, :N] + degp[1, :N] + 1.0)[:, None]
    hp = _tc_hprime(x, kernel, deg_col)
    aggp = _sc_aggregate(col, row, hp)
    return _tc_final(aggp[:, :N, :], hp, deg_col, bias[None, :])


# speedup vs baseline: 33.6170x; 1.8920x over previous
import functools

import jax
import jax.numpy as jnp
from jax import lax
from jax.experimental import pallas as pl
from jax.experimental.pallas import tpu as pltpu
from jax.experimental.pallas import tpu_sc as plsc

N = 10000
E = 320000
D = 128
NCORES = 2
NSUB = 16
NW = NCORES * NSUB
EPT = E // NW
PAD_N = 10240
STRIPE = PAD_N // NSUB
RB = 32

NSB = 5

CH_A = 40
NCH_A = EPT // CH_A
NB_A = 2
NG_A = NCH_A // NB_A
SB_A = NCH_A // NSB
GSB_A = NG_A // NSB

CH_D = 80
NCH_D = EPT // CH_D
NB_D = 5
NG_D = NCH_D // NB_D
SB_D = NCH_D // NSB
GSB_D = NG_D // NSB

_MESH = plsc.VectorSubcoreMesh(core_axis_name="c", subcore_axis_name="s")


def _sc_degree(row3):
    @functools.partial(
        pl.kernel,
        out_type=jax.ShapeDtypeStruct((NCORES, PAD_N), jnp.float32),
        mesh=_MESH,
        scratch_types=[
            pltpu.VMEM((2, SB_D, CH_D), jnp.int32),
            pltpu.VMEM((CH_D,), jnp.float32),
            pltpu.VMEM((STRIPE,), jnp.float32),
            pltpu.VMEM_SHARED((PAD_N,), jnp.float32),
            [pltpu.SemaphoreType.DMA for _ in range(NB_D)],
            pltpu.SemaphoreType.DMA,
        ],
    )
    def k(row_hbm, out_hbm, idx_v, ones_v, buf_v, deg_sh, ssems, isem):
        c = lax.axis_index("c")
        s = lax.axis_index("s")
        w = c * NSUB + s
        pltpu.sync_copy(row_hbm.at[w, 0], idx_v.at[0])
        one16 = jnp.ones((16,), jnp.float32)
        zero16 = jnp.zeros((16,), jnp.float32)
        for i in range(CH_D // 16):
            ones_v[pl.ds(i * 16, 16)] = one16

        def zb(i, _):
            buf_v[pl.ds(i * 16, 16)] = zero16
            return 0

        lax.fori_loop(0, STRIPE // 16, zb, 0)
        pltpu.sync_copy(buf_v, deg_sh.at[pl.ds(s * STRIPE, STRIPE)])
        plsc.subcore_barrier()

        def body(g, _):
            sb = g // GSB_D
            rg = g - sb * GSB_D
            slot = lax.rem(sb, 2)

            @pl.when((rg == 0) & (sb > 0))
            def _wait_idx():
                pltpu.make_async_copy(row_hbm.at[w, sb], idx_v.at[slot], isem).wait()

            @pl.when((rg == 1) & (sb < NSB - 1))
            def _load_idx():
                nslot = lax.rem(sb + 1, 2)
                pltpu.async_copy(row_hbm.at[w, sb + 1], idx_v.at[nslot], isem)

            for b in range(NB_D):
                row = rg * NB_D + b

                @pl.when(g > 0)
                def _drain(b=b, g=g):
                    gp = g - 1
                    sbp = gp // GSB_D
                    rowp = (gp - sbp * GSB_D) * NB_D + b
                    slotp = lax.rem(sbp, 2)
                    pltpu.make_async_copy(
                        ones_v, deg_sh.at[idx_v.at[slotp, rowp]], ssems[b]
                    ).wait()

                pltpu.async_copy(
                    ones_v, deg_sh.at[idx_v.at[slot, row]], ssems[b], add=True
                )
            return 0

        lax.fori_loop(0, NG_D, body, 0)
        for b in range(NB_D):
            row = (GSB_D - 1) * NB_D + b
            slot = (NSB - 1) % 2
            pltpu.make_async_copy(
                ones_v, deg_sh.at[idx_v.at[slot, row]], ssems[b]
            ).wait()
        plsc.subcore_barrier()
        pltpu.sync_copy(deg_sh.at[pl.ds(s * STRIPE, STRIPE)], buf_v)
        pltpu.sync_copy(buf_v, out_hbm.at[c, pl.ds(s * STRIPE, STRIPE)])

    return k(row3)


def _sc_aggregate(col4, row4, hp):
    @functools.partial(
        pl.kernel,
        out_type=jax.ShapeDtypeStruct((NCORES, PAD_N, D), jnp.float32),
        mesh=_MESH,
        scratch_types=[
            pltpu.VMEM((2, SB_A, CH_A), jnp.int32),
            pltpu.VMEM((2, SB_A, CH_A), jnp.int32),
            [pltpu.VMEM((CH_A, D), jnp.float32) for _ in range(NB_A)],
            pltpu.VMEM((RB, D), jnp.float32),
            pltpu.VMEM_SHARED((PAD_N, D), jnp.float32),
            [pltpu.SemaphoreType.DMA for _ in range(NB_A)],
            [pltpu.SemaphoreType.DMA for _ in range(NB_A)],
            pltpu.SemaphoreType.DMA,
            pltpu.SemaphoreType.DMA,
        ],
    )
    def k(col_hbm, row_hbm, hp_hbm, out_hbm, ci_v, ri_v, bufs, stg_v, agg_sh,
          gsems, ssems, icsem, irsem):
        c = lax.axis_index("c")
        s = lax.axis_index("s")
        w = c * NSUB + s
        pltpu.sync_copy(col_hbm.at[w, 0], ci_v.at[0])
        pltpu.sync_copy(row_hbm.at[w, 0], ri_v.at[0])
        zero16 = jnp.zeros((16,), jnp.float32)

        def zb(i, _):
            r = i // (D // 16)
            q = i % (D // 16)
            stg_v[r, pl.ds(q * 16, 16)] = zero16
            return 0

        lax.fori_loop(0, RB * (D // 16), zb, 0)
        for t in range(STRIPE // RB):
            pltpu.sync_copy(stg_v, agg_sh.at[pl.ds(s * STRIPE + t * RB, RB)])
        plsc.subcore_barrier()

        for b in range(NB_A):
            pltpu.async_copy(hp_hbm.at[ci_v.at[0, b]], bufs[b], gsems[b])

        def body(g, _):
            sb = g // GSB_A
            rg = g - sb * GSB_A
            slot = lax.rem(sb, 2)

            @pl.when((rg == 1) & (sb < NSB - 1))
            def _load_idx():
                nslot = lax.rem(sb + 1, 2)
                pltpu.async_copy(col_hbm.at[w, sb + 1], ci_v.at[nslot], icsem)
                pltpu.async_copy(row_hbm.at[w, sb + 1], ri_v.at[nslot], irsem)

            @pl.when((rg == GSB_A - 1) & (sb < NSB - 1))
            def _wait_idx():
                nslot = lax.rem(sb + 1, 2)
                pltpu.make_async_copy(
                    col_hbm.at[w, sb + 1], ci_v.at[nslot], icsem
                ).wait()
                pltpu.make_async_copy(
                    row_hbm.at[w, sb + 1], ri_v.at[nslot], irsem
                ).wait()

            for b in range(NB_A):
                row = rg * NB_A + b
                pltpu.make_async_copy(
                    hp_hbm.at[ci_v.at[slot, row]], bufs[b], gsems[b]
                ).wait()
                pltpu.async_copy(
                    bufs[b], agg_sh.at[ri_v.at[slot, row]], ssems[b], add=True
                )
                pltpu.make_async_copy(
                    bufs[b], agg_sh.at[ri_v.at[slot, row]], ssems[b]
                ).wait()

                @pl.when(g < NG_A - 1)
                def _prefetch(b=b, g=g):
                    gn = g + 1
                    sbn = gn // GSB_A
                    rown = (gn - sbn * GSB_A) * NB_A + b
                    slotn = lax.rem(sbn, 2)
                    pltpu.async_copy(
                        hp_hbm.at[ci_v.at[slotn, rown]], bufs[b], gsems[b]
                    )

            return 0

        lax.fori_loop(0, NG_A, body, 0)
        plsc.subcore_barrier()
        pltpu.sync_copy(
            agg_sh.at[pl.ds(s * STRIPE, STRIPE)],
            out_hbm.at[c, pl.ds(s * STRIPE, STRIPE)],
        )

    return k(col4, row4, hp)


_M_BLK = 1000


def _tc_hprime(x, w, deg_col):
    def body(x_ref, w_ref, d_ref, o_ref):
        dis = lax.rsqrt(d_ref[...])
        h = jnp.dot(x_ref[...], w_ref[...], preferred_element_type=jnp.float32)
        o_ref[...] = dis * h

    return pl.pallas_call(
        body,
        grid=(N // _M_BLK,),
        in_specs=[
            pl.BlockSpec((_M_BLK, D), lambda i: (i, 0)),
            pl.BlockSpec((D, D), lambda i: (0, 0)),
            pl.BlockSpec((_M_BLK, 1), lambda i: (i, 0)),
        ],
        out_specs=pl.BlockSpec((_M_BLK, D), lambda i: (i, 0)),
        out_shape=jax.ShapeDtypeStruct((N, D), jnp.float32),
    )(x, w, deg_col)


def _tc_final(aggp, hp, deg_col, bias2d):
    def body(a_ref, h_ref, d_ref, b_ref, o_ref):
        dis = lax.rsqrt(d_ref[...])
        o_ref[...] = dis * (a_ref[0] + a_ref[1] + h_ref[...]) + b_ref[...]

    return pl.pallas_call(
        body,
        grid=(N // _M_BLK,),
        in_specs=[
            pl.BlockSpec((NCORES, _M_BLK, D), lambda i: (0, i, 0)),
            pl.BlockSpec((_M_BLK, D), lambda i: (i, 0)),
            pl.BlockSpec((_M_BLK, 1), lambda i: (i, 0)),
            pl.BlockSpec((1, D), lambda i: (0, 0)),
        ],
        out_specs=pl.BlockSpec((_M_BLK, D), lambda i: (i, 0)),
        out_shape=jax.ShapeDtypeStruct((N, D), jnp.float32),
    )(aggp, hp, deg_col, bias2d)


def kernel(x, edge_index, kernel, bias):
    row3 = edge_index[0].reshape(NW, NSB, SB_D, CH_D)
    row4 = edge_index[0].reshape(NW, NSB, SB_A, CH_A)
    col4 = edge_index[1].reshape(NW, NSB, SB_A, CH_A)
    degp = _sc_degree(row3)
    deg_col = (degp[0, :N] + degp[1, :N] + 1.0)[:, None]
    hp = _tc_hprime(x, kernel, deg_col)
    aggp = _sc_aggregate(col4, row4, hp)
    return _tc_final(aggp[:, :N, :], hp, deg_col, bias[None, :])


# speedup vs baseline: 47.3521x; 1.4086x over previous
import functools

import jax
import jax.numpy as jnp
from jax import lax
from jax.experimental import pallas as pl
from jax.experimental.pallas import tpu as pltpu
from jax.experimental.pallas import tpu_sc as plsc

N = 10000
E = 320000
D = 128
NCORES = 2
NSUB = 16
NW = NCORES * NSUB
EPT = E // NW
PAD_A = 10112
STRIPE_A = PAD_A // NSUB
PAD_D = 10240
STRIPE_D = PAD_D // NSUB
RB = 8

NSB = 10

CH_A = 40
NCH_A = EPT // CH_A
NB_A = 5
NG_A = NCH_A // NB_A
SB_A = NCH_A // NSB
GSB_A = NG_A // NSB

CH_D = 80
NCH_D = EPT // CH_D
NB_D = 5
NG_D = NCH_D // NB_D
NSB_D = 5
SB_D = NCH_D // NSB_D
GSB_D = NG_D // NSB_D

_MESH = plsc.VectorSubcoreMesh(core_axis_name="c", subcore_axis_name="s")


def _sc_degree(row3):
    @functools.partial(
        pl.kernel,
        out_type=jax.ShapeDtypeStruct((NCORES, PAD_D), jnp.float32),
        mesh=_MESH,
        scratch_types=[
            pltpu.VMEM((2, SB_D, CH_D), jnp.int32),
            pltpu.VMEM((CH_D,), jnp.float32),
            pltpu.VMEM((STRIPE_D,), jnp.float32),
            pltpu.VMEM_SHARED((PAD_D,), jnp.float32),
            [pltpu.SemaphoreType.DMA for _ in range(NB_D)],
            pltpu.SemaphoreType.DMA,
        ],
    )
    def k(row_hbm, out_hbm, idx_v, ones_v, buf_v, deg_sh, ssems, isem):
        c = lax.axis_index("c")
        s = lax.axis_index("s")
        w = c * NSUB + s
        pltpu.sync_copy(row_hbm.at[w, 0], idx_v.at[0])
        one16 = jnp.ones((16,), jnp.float32)
        zero16 = jnp.zeros((16,), jnp.float32)
        for i in range(CH_D // 16):
            ones_v[pl.ds(i * 16, 16)] = one16

        def zb(i, _):
            buf_v[pl.ds(i * 16, 16)] = zero16
            return 0

        lax.fori_loop(0, STRIPE_D // 16, zb, 0)
        pltpu.sync_copy(buf_v, deg_sh.at[pl.ds(s * STRIPE_D, STRIPE_D)])
        plsc.subcore_barrier()

        def body(g, _):
            sb = g // GSB_D
            rg = g - sb * GSB_D
            slot = lax.rem(sb, 2)

            @pl.when((rg == 0) & (sb > 0))
            def _wait_idx():
                pltpu.make_async_copy(row_hbm.at[w, sb], idx_v.at[slot], isem).wait()

            @pl.when((rg == 1) & (sb < NSB_D - 1))
            def _load_idx():
                nslot = lax.rem(sb + 1, 2)
                pltpu.async_copy(row_hbm.at[w, sb + 1], idx_v.at[nslot], isem)

            for b in range(NB_D):
                row = rg * NB_D + b

                @pl.when(g > 0)
                def _drain(b=b, g=g):
                    gp = g - 1
                    sbp = gp // GSB_D
                    rowp = (gp - sbp * GSB_D) * NB_D + b
                    slotp = lax.rem(sbp, 2)
                    pltpu.make_async_copy(
                        ones_v, deg_sh.at[idx_v.at[slotp, rowp]], ssems[b]
                    ).wait()

                pltpu.async_copy(
                    ones_v, deg_sh.at[idx_v.at[slot, row]], ssems[b], add=True
                )
            return 0

        lax.fori_loop(0, NG_D, body, 0)
        for b in range(NB_D):
            row = (GSB_D - 1) * NB_D + b
            slot = (NSB_D - 1) % 2
            pltpu.make_async_copy(
                ones_v, deg_sh.at[idx_v.at[slot, row]], ssems[b]
            ).wait()
        plsc.subcore_barrier()
        pltpu.sync_copy(deg_sh.at[pl.ds(s * STRIPE_D, STRIPE_D)], buf_v)
        pltpu.sync_copy(buf_v, out_hbm.at[c, pl.ds(s * STRIPE_D, STRIPE_D)])

    return k(row3)


def _sc_aggregate(col4, row4, hp):
    @functools.partial(
        pl.kernel,
        out_type=jax.ShapeDtypeStruct((NCORES, PAD_A, D), jnp.float32),
        mesh=_MESH,
        scratch_types=[
            pltpu.VMEM((2, SB_A, CH_A), jnp.int32),
            pltpu.VMEM((2, SB_A, CH_A), jnp.int32),
            [pltpu.VMEM((CH_A, D), jnp.float32) for _ in range(NB_A)],
            pltpu.VMEM((RB, D), jnp.float32),
            pltpu.VMEM_SHARED((PAD_A, D), jnp.float32),
            [pltpu.SemaphoreType.DMA for _ in range(NB_A)],
            [pltpu.SemaphoreType.DMA for _ in range(NB_A)],
            pltpu.SemaphoreType.DMA,
            pltpu.SemaphoreType.DMA,
        ],
    )
    def k(col_hbm, row_hbm, hp_hbm, out_hbm, ci_v, ri_v, bufs, stg_v, agg_sh,
          gsems, ssems, icsem, irsem):
        c = lax.axis_index("c")
        s = lax.axis_index("s")
        w = c * NSUB + s
        pltpu.sync_copy(col_hbm.at[w, 0], ci_v.at[0])
        pltpu.sync_copy(row_hbm.at[w, 0], ri_v.at[0])
        zero16 = jnp.zeros((16,), jnp.float32)

        def zb(i, _):
            r = i // (D // 16)
            q = i % (D // 16)
            stg_v[r, pl.ds(q * 16, 16)] = zero16
            return 0

        lax.fori_loop(0, RB * (D // 16), zb, 0)
        for t in range(STRIPE_A // RB):
            pltpu.sync_copy(stg_v, agg_sh.at[pl.ds(s * STRIPE_A + t * RB, RB)])
        plsc.subcore_barrier()

        for b in range(NB_A):
            pltpu.async_copy(hp_hbm.at[ci_v.at[0, b]], bufs[b], gsems[b])

        def body(g, _):
            sb = g // GSB_A
            rg = g - sb * GSB_A
            slot = lax.rem(sb, 2)

            @pl.when((rg == 1) & (sb < NSB - 1))
            def _load_idx():
                nslot = lax.rem(sb + 1, 2)
                pltpu.async_copy(col_hbm.at[w, sb + 1], ci_v.at[nslot], icsem)
                pltpu.async_copy(row_hbm.at[w, sb + 1], ri_v.at[nslot], irsem)

            @pl.when((rg == GSB_A - 1) & (sb < NSB - 1))
            def _wait_idx():
                nslot = lax.rem(sb + 1, 2)
                pltpu.make_async_copy(
                    col_hbm.at[w, sb + 1], ci_v.at[nslot], icsem
                ).wait()
                pltpu.make_async_copy(
                    row_hbm.at[w, sb + 1], ri_v.at[nslot], irsem
                ).wait()

            for b in range(NB_A):
                row = rg * NB_A + b
                pltpu.make_async_copy(
                    hp_hbm.at[ci_v.at[slot, row]], bufs[b], gsems[b]
                ).wait()
                pltpu.async_copy(
                    bufs[b], agg_sh.at[ri_v.at[slot, row]], ssems[b], add=True
                )
                pltpu.make_async_copy(
                    bufs[b], agg_sh.at[ri_v.at[slot, row]], ssems[b]
                ).wait()

                @pl.when(g < NG_A - 1)
                def _prefetch(b=b, g=g):
                    gn = g + 1
                    sbn = gn // GSB_A
                    rown = (gn - sbn * GSB_A) * NB_A + b
                    slotn = lax.rem(sbn, 2)
                    pltpu.async_copy(
                        hp_hbm.at[ci_v.at[slotn, rown]], bufs[b], gsems[b]
                    )

            return 0

        lax.fori_loop(0, NG_A, body, 0)
        plsc.subcore_barrier()
        pltpu.sync_copy(
            agg_sh.at[pl.ds(s * STRIPE_A, STRIPE_A)],
            out_hbm.at[c, pl.ds(s * STRIPE_A, STRIPE_A)],
        )

    return k(col4, row4, hp)


_M_BLK = 1000


def _tc_hprime(x, w, deg_col):
    def body(x_ref, w_ref, d_ref, o_ref):
        dis = lax.rsqrt(d_ref[...])
        h = jnp.dot(x_ref[...], w_ref[...], preferred_element_type=jnp.float32)
        o_ref[...] = dis * h

    return pl.pallas_call(
        body,
        grid=(N // _M_BLK,),
        in_specs=[
            pl.BlockSpec((_M_BLK, D), lambda i: (i, 0)),
            pl.BlockSpec((D, D), lambda i: (0, 0)),
            pl.BlockSpec((_M_BLK, 1), lambda i: (i, 0)),
        ],
        out_specs=pl.BlockSpec((_M_BLK, D), lambda i: (i, 0)),
        out_shape=jax.ShapeDtypeStruct((N, D), jnp.float32),
    )(x, w, deg_col)


def _tc_final(aggp, hp, deg_col, bias2d):
    def body(a_ref, h_ref, d_ref, b_ref, o_ref):
        dis = lax.rsqrt(d_ref[...])
        o_ref[...] = dis * (a_ref[0] + a_ref[1] + h_ref[...]) + b_ref[...]

    return pl.pallas_call(
        body,
        grid=(N // _M_BLK,),
        in_specs=[
            pl.BlockSpec((NCORES, _M_BLK, D), lambda i: (0, i, 0)),
            pl.BlockSpec((_M_BLK, D), lambda i: (i, 0)),
            pl.BlockSpec((_M_BLK, 1), lambda i: (i, 0)),
            pl.BlockSpec((1, D), lambda i: (0, 0)),
        ],
        out_specs=pl.BlockSpec((_M_BLK, D), lambda i: (i, 0)),
        out_shape=jax.ShapeDtypeStruct((N, D), jnp.float32),
    )(aggp, hp, deg_col, bias2d)


def kernel(x, edge_index, kernel, bias):
    row3 = edge_index[0].reshape(NW, NSB_D, SB_D, CH_D)
    row4 = edge_index[0].reshape(NW, NSB, SB_A, CH_A)
    col4 = edge_index[1].reshape(NW, NSB, SB_A, CH_A)
    degp = _sc_degree(row3)
    deg_col = (degp[0, :N] + degp[1, :N] + 1.0)[:, None]
    hp = _tc_hprime(x, kernel, deg_col)
    aggp = _sc_aggregate(col4, row4, hp)
    return _tc_final(aggp, hp, deg_col, bias[None, :])


# speedup vs baseline: 48.6005x; 1.0264x over previous
import functools

import jax
import jax.numpy as jnp
from jax import lax
from jax.experimental import pallas as pl
from jax.experimental.pallas import tpu as pltpu
from jax.experimental.pallas import tpu_sc as plsc

N = 10000
E = 320000
D = 128
NCORES = 2
NSUB = 16
NW = NCORES * NSUB
EPT = E // NW
PAD_A = 10112
STRIPE_A = PAD_A // NSUB
PAD_D = 10240
STRIPE_D = PAD_D // NSUB
RB = 8

NSB = 10

CH_A = 40
NCH_A = EPT // CH_A
NB_A = 5
NG_A = NCH_A // NB_A
SB_A = NCH_A // NSB
GSB_A = NG_A // NSB

CH_D = 80
NCH_D = EPT // CH_D
NB_D = 5
NG_D = NCH_D // NB_D
NSB_D = 5
SB_D = NCH_D // NSB_D
GSB_D = NG_D // NSB_D

_MESH = plsc.VectorSubcoreMesh(core_axis_name="c", subcore_axis_name="s")


SBW_D = SB_D * CH_D
SBW_A = SB_A * CH_A


def _sc_degree(row3):
    @functools.partial(
        pl.kernel,
        out_type=jax.ShapeDtypeStruct((NCORES, PAD_D), jnp.float32),
        mesh=_MESH,
        scratch_types=[
            [pltpu.VMEM((SBW_D,), jnp.int32) for _ in range(2)],
            pltpu.VMEM((CH_D,), jnp.float32),
            pltpu.VMEM((STRIPE_D,), jnp.float32),
            pltpu.VMEM_SHARED((PAD_D,), jnp.float32),
            [pltpu.SemaphoreType.DMA for _ in range(NB_D)],
            pltpu.SemaphoreType.DMA,
        ],
    )
    def k(ei_hbm, out_hbm, idxs, ones_v, buf_v, deg_sh, ssems, isem):
        c = lax.axis_index("c")
        s = lax.axis_index("s")
        w = c * NSUB + s
        pltpu.sync_copy(ei_hbm.at[w * NSB_D, 0], idxs[0])
        one16 = jnp.ones((16,), jnp.float32)
        zero16 = jnp.zeros((16,), jnp.float32)
        for i in range(CH_D // 16):
            ones_v[pl.ds(i * 16, 16)] = one16

        def zb(i, _):
            buf_v[pl.ds(i * 16, 16)] = zero16
            return 0

        lax.fori_loop(0, STRIPE_D // 16, zb, 0)
        pltpu.sync_copy(buf_v, deg_sh.at[pl.ds(s * STRIPE_D, STRIPE_D)])
        plsc.subcore_barrier()

        for sb in range(NSB_D):
            cur = idxs[sb % 2]
            prv = idxs[(sb - 1) % 2]
            nxt = idxs[(sb + 1) % 2]
            if sb > 0:
                pltpu.make_async_copy(ei_hbm.at[w * NSB_D + sb, 0], cur, isem).wait()

            def body(g, _, sb=sb, cur=cur, prv=prv, nxt=nxt):
                if sb < NSB_D - 1:

                    @pl.when(g == 1)
                    def _load_idx():
                        pltpu.async_copy(ei_hbm.at[w * NSB_D + sb + 1, 0], nxt, isem)

                for b in range(NB_D):
                    row = g * NB_D + b
                    if sb == 0:

                        @pl.when(g > 0)
                        def _drain(b=b):
                            rowp = (g - 1) * NB_D + b
                            pltpu.make_async_copy(
                                ones_v,
                                deg_sh.at[cur.at[pl.ds(rowp * CH_D, CH_D)]],
                                ssems[b],
                            ).wait()

                    else:

                        @pl.when(g > 0)
                        def _drain_same(b=b):
                            rowp = (g - 1) * NB_D + b
                            pltpu.make_async_copy(
                                ones_v,
                                deg_sh.at[cur.at[pl.ds(rowp * CH_D, CH_D)]],
                                ssems[b],
                            ).wait()

                        @pl.when(g == 0)
                        def _drain_prev(b=b):
                            rowp = (GSB_D - 1) * NB_D + b
                            pltpu.make_async_copy(
                                ones_v,
                                deg_sh.at[prv.at[pl.ds(rowp * CH_D, CH_D)]],
                                ssems[b],
                            ).wait()

                    pltpu.async_copy(
                        ones_v,
                        deg_sh.at[cur.at[pl.ds(row * CH_D, CH_D)]],
                        ssems[b], add=True,
                    )
                return 0

            lax.fori_loop(0, GSB_D, body, 0)

        last = idxs[(NSB_D - 1) % 2]
        for b in range(NB_D):
            rowp = (GSB_D - 1) * NB_D + b
            pltpu.make_async_copy(
                ones_v, deg_sh.at[last.at[pl.ds(rowp * CH_D, CH_D)]], ssems[b]
            ).wait()
        plsc.subcore_barrier()
        pltpu.sync_copy(deg_sh.at[pl.ds(s * STRIPE_D, STRIPE_D)], buf_v)
        pltpu.sync_copy(buf_v, out_hbm.at[c, pl.ds(s * STRIPE_D, STRIPE_D)])

    return k(row3)


def _sc_aggregate(rowa, cola, hp):
    @functools.partial(
        pl.kernel,
        out_type=jax.ShapeDtypeStruct((NCORES, PAD_A, D), jnp.float32),
        mesh=_MESH,
        scratch_types=[
            [pltpu.VMEM((SBW_A,), jnp.int32) for _ in range(2)],
            [pltpu.VMEM((SBW_A,), jnp.int32) for _ in range(2)],
            [pltpu.VMEM((CH_A, D), jnp.float32) for _ in range(NB_A)],
            pltpu.VMEM((RB, D), jnp.float32),
            pltpu.VMEM_SHARED((PAD_A, D), jnp.float32),
            [pltpu.SemaphoreType.DMA for _ in range(NB_A)],
            [pltpu.SemaphoreType.DMA for _ in range(NB_A)],
            pltpu.SemaphoreType.DMA,
            pltpu.SemaphoreType.DMA,
        ],
    )
    def k(row_hbm, col_hbm, hp_hbm, out_hbm, cis, ris, bufs, stg_v, agg_sh,
          gsems, ssems, icsem, irsem):
        c = lax.axis_index("c")
        s = lax.axis_index("s")
        w = c * NSUB + s
        pltpu.sync_copy(col_hbm.at[w * NSB, 0], cis[0])
        pltpu.sync_copy(row_hbm.at[w * NSB, 0], ris[0])
        zero16 = jnp.zeros((16,), jnp.float32)

        def zb(i, _):
            r = i // (D // 16)
            q = i % (D // 16)
            stg_v[r, pl.ds(q * 16, 16)] = zero16
            return 0

        lax.fori_loop(0, RB * (D // 16), zb, 0)
        for t in range(STRIPE_A // RB):
            pltpu.sync_copy(stg_v, agg_sh.at[pl.ds(s * STRIPE_A + t * RB, RB)])
        plsc.subcore_barrier()

        for b in range(NB_A):
            pltpu.async_copy(
                hp_hbm.at[cis[0].at[pl.ds(b * CH_A, CH_A)]], bufs[b], gsems[b]
            )

        for sb in range(NSB):
            ccur, rcur = cis[sb % 2], ris[sb % 2]
            cnxt, rnxt = cis[(sb + 1) % 2], ris[(sb + 1) % 2]

            def body(g, _, sb=sb, ccur=ccur, rcur=rcur, cnxt=cnxt, rnxt=rnxt):
                if sb < NSB - 1:

                    @pl.when(g == 1)
                    def _load_idx():
                        pltpu.async_copy(col_hbm.at[w * NSB + sb + 1, 0], cnxt, icsem)
                        pltpu.async_copy(row_hbm.at[w * NSB + sb + 1, 0], rnxt, irsem)

                    @pl.when(g == GSB_A - 1)
                    def _wait_idx():
                        pltpu.make_async_copy(
                            col_hbm.at[w * NSB + sb + 1, 0], cnxt, icsem
                        ).wait()
                        pltpu.make_async_copy(
                            row_hbm.at[w * NSB + sb + 1, 0], rnxt, irsem
                        ).wait()

                for b in range(NB_A):
                    row = g * NB_A + b
                    pltpu.make_async_copy(
                        hp_hbm.at[ccur.at[pl.ds(row * CH_A, CH_A)]],
                        bufs[b], gsems[b],
                    ).wait()
                    pltpu.async_copy(
                        bufs[b],
                        agg_sh.at[rcur.at[pl.ds(row * CH_A, CH_A)]],
                        ssems[b], add=True,
                    )
                    pltpu.make_async_copy(
                        bufs[b],
                        agg_sh.at[rcur.at[pl.ds(row * CH_A, CH_A)]],
                        ssems[b],
                    ).wait()

                    if sb < NSB - 1:

                        @pl.when(g < GSB_A - 1)
                        def _pf_same(b=b):
                            rown = (g + 1) * NB_A + b
                            pltpu.async_copy(
                                hp_hbm.at[ccur.at[pl.ds(rown * CH_A, CH_A)]],
                                bufs[b], gsems[b],
                            )

                        @pl.when(g == GSB_A - 1)
                        def _pf_next(b=b):
                            pltpu.async_copy(
                                hp_hbm.at[cnxt.at[pl.ds(b * CH_A, CH_A)]],
                                bufs[b], gsems[b],
                            )

                    else:

                        @pl.when(g < GSB_A - 1)
                        def _pf_same(b=b):
                            rown = (g + 1) * NB_A + b
                            pltpu.async_copy(
                                hp_hbm.at[ccur.at[pl.ds(rown * CH_A, CH_A)]],
                                bufs[b], gsems[b],
                            )

                return 0

            lax.fori_loop(0, GSB_A, body, 0)

        plsc.subcore_barrier()
        pltpu.sync_copy(
            agg_sh.at[pl.ds(s * STRIPE_A, STRIPE_A)],
            out_hbm.at[c, pl.ds(s * STRIPE_A, STRIPE_A)],
        )

    return k(rowa, cola, hp)


_M_BLK = 1000


def _tc_hprime(x, w, deg_col):
    def body(x_ref, w_ref, d_ref, o_ref):
        dis = lax.rsqrt(d_ref[...])
        h = jnp.dot(x_ref[...], w_ref[...], preferred_element_type=jnp.float32)
        o_ref[...] = dis * h

    return pl.pallas_call(
        body,
        grid=(N // _M_BLK,),
        in_specs=[
            pl.BlockSpec((_M_BLK, D), lambda i: (i, 0)),
            pl.BlockSpec((D, D), lambda i: (0, 0)),
            pl.BlockSpec((_M_BLK, 1), lambda i: (i, 0)),
        ],
        out_specs=pl.BlockSpec((_M_BLK, D), lambda i: (i, 0)),
        out_shape=jax.ShapeDtypeStruct((N, D), jnp.float32),
    )(x, w, deg_col)


def _tc_final(aggp, hp, deg_col, bias2d):
    def body(a_ref, h_ref, d_ref, b_ref, o_ref):
        dis = lax.rsqrt(d_ref[...])
        o_ref[...] = dis * (a_ref[0] + a_ref[1] + h_ref[...]) + b_ref[...]

    return pl.pallas_call(
        body,
        grid=(N // _M_BLK,),
        in_specs=[
            pl.BlockSpec((NCORES, _M_BLK, D), lambda i: (0, i, 0)),
            pl.BlockSpec((_M_BLK, D), lambda i: (i, 0)),
            pl.BlockSpec((_M_BLK, 1), lambda i: (i, 0)),
            pl.BlockSpec((1, D), lambda i: (0, 0)),
        ],
        out_specs=pl.BlockSpec((_M_BLK, D), lambda i: (i, 0)),
        out_shape=jax.ShapeDtypeStruct((N, D), jnp.float32),
    )(aggp, hp, deg_col, bias2d)


def kernel(x, edge_index, kernel, bias):
    row3 = edge_index[0].reshape(NW * NSB_D, 1, SBW_D)
    rowa = edge_index[0].reshape(NW * NSB, 1, SBW_A)
    cola = edge_index[1].reshape(NW * NSB, 1, SBW_A)
    degp = _sc_degree(row3)
    deg_col = (degp[0, :N] + degp[1, :N] + 1.0)[:, None]
    hp = _tc_hprime(x, kernel, deg_col)
    aggp = _sc_aggregate(rowa, cola, hp)
    return _tc_final(aggp, hp, deg_col, bias[None, :])


# speedup vs baseline: 50.7576x; 1.0444x over previous
import functools

import jax
import jax.numpy as jnp
from jax import lax
from jax.experimental import pallas as pl
from jax.experimental.pallas import tpu as pltpu
from jax.experimental.pallas import tpu_sc as plsc

N = 10000
E = 320000
D = 128
NCORES = 2
NSUB = 16
NW = NCORES * NSUB
EPT = E // NW
PAD_A = 10112
STRIPE_A = PAD_A // NSUB
PAD_D = 10240
STRIPE_D = PAD_D // NSUB
RB = 8

NSB = 10

CH_A = 40
NCH_A = EPT // CH_A
NB_A = 5
NG_A = NCH_A // NB_A
SB_A = NCH_A // NSB
GSB_A = NG_A // NSB

CH_D = 80
NCH_D = EPT // CH_D
NB_D = 5
NG_D = NCH_D // NB_D
NSB_D = 5
SB_D = NCH_D // NSB_D
GSB_D = NG_D // NSB_D

_MESH = plsc.VectorSubcoreMesh(core_axis_name="c", subcore_axis_name="s")


SBW_D = SB_D * CH_D
SBW_A = SB_A * CH_A


def _sc_degree(ei4d):
    @functools.partial(
        pl.kernel,
        out_type=jax.ShapeDtypeStruct((NCORES, PAD_D), jnp.float32),
        mesh=_MESH,
        scratch_types=[
            [pltpu.VMEM((SBW_D,), jnp.int32) for _ in range(2)],
            pltpu.VMEM((CH_D,), jnp.float32),
            pltpu.VMEM((STRIPE_D,), jnp.float32),
            pltpu.VMEM_SHARED((PAD_D,), jnp.float32),
            [pltpu.SemaphoreType.DMA for _ in range(NB_D)],
            pltpu.SemaphoreType.DMA,
        ],
    )
    def k(ei_hbm, out_hbm, idxs, ones_v, buf_v, deg_sh, ssems, isem):
        c = lax.axis_index("c")
        s = lax.axis_index("s")
        w = c * NSUB + s
        pltpu.sync_copy(ei_hbm.at[0, w * NSB_D, 0], idxs[0])
        one16 = jnp.ones((16,), jnp.float32)
        zero16 = jnp.zeros((16,), jnp.float32)
        for i in range(CH_D // 16):
            ones_v[pl.ds(i * 16, 16)] = one16

        def zb(i, _):
            buf_v[pl.ds(i * 16, 16)] = zero16
            return 0

        lax.fori_loop(0, STRIPE_D // 16, zb, 0)
        pltpu.sync_copy(buf_v, deg_sh.at[pl.ds(s * STRIPE_D, STRIPE_D)])
        plsc.subcore_barrier()

        for sb in range(NSB_D):
            cur = idxs[sb % 2]
            prv = idxs[(sb - 1) % 2]
            nxt = idxs[(sb + 1) % 2]
            if sb > 0:
                pltpu.make_async_copy(ei_hbm.at[0, w * NSB_D + sb, 0], cur, isem).wait()

            def body(g, _, sb=sb, cur=cur, prv=prv, nxt=nxt):
                if sb < NSB_D - 1:

                    @pl.when(g == 1)
                    def _load_idx():
                        pltpu.async_copy(ei_hbm.at[0, w * NSB_D + sb + 1, 0], nxt, isem)

                for b in range(NB_D):
                    row = g * NB_D + b
                    if sb == 0:

                        @pl.when(g > 0)
                        def _drain(b=b):
                            rowp = (g - 1) * NB_D + b
                            pltpu.make_async_copy(
                                ones_v,
                                deg_sh.at[cur.at[pl.ds(rowp * CH_D, CH_D)]],
                                ssems[b],
                            ).wait()

                    else:

                        @pl.when(g > 0)
                        def _drain_same(b=b):
                            rowp = (g - 1) * NB_D + b
                            pltpu.make_async_copy(
                                ones_v,
                                deg_sh.at[cur.at[pl.ds(rowp * CH_D, CH_D)]],
                                ssems[b],
                            ).wait()

                        @pl.when(g == 0)
                        def _drain_prev(b=b):
                            rowp = (GSB_D - 1) * NB_D + b
                            pltpu.make_async_copy(
                                ones_v,
                                deg_sh.at[prv.at[pl.ds(rowp * CH_D, CH_D)]],
                                ssems[b],
                            ).wait()

                    pltpu.async_copy(
                        ones_v,
                        deg_sh.at[cur.at[pl.ds(row * CH_D, CH_D)]],
                        ssems[b], add=True,
                    )
                return 0

            lax.fori_loop(0, GSB_D, body, 0)

        last = idxs[(NSB_D - 1) % 2]
        for b in range(NB_D):
            rowp = (GSB_D - 1) * NB_D + b
            pltpu.make_async_copy(
                ones_v, deg_sh.at[last.at[pl.ds(rowp * CH_D, CH_D)]], ssems[b]
            ).wait()
        plsc.subcore_barrier()
        pltpu.sync_copy(deg_sh.at[pl.ds(s * STRIPE_D, STRIPE_D)], buf_v)
        pltpu.sync_copy(buf_v, out_hbm.at[c, pl.ds(s * STRIPE_D, STRIPE_D)])

    return k(ei4d)


def _sc_aggregate(ei4a, hp):
    @functools.partial(
        pl.kernel,
        out_type=jax.ShapeDtypeStruct((NCORES, PAD_A, D), jnp.float32),
        mesh=_MESH,
        scratch_types=[
            [pltpu.VMEM((SBW_A,), jnp.int32) for _ in range(2)],
            [pltpu.VMEM((SBW_A,), jnp.int32) for _ in range(2)],
            [pltpu.VMEM((CH_A, D), jnp.float32) for _ in range(NB_A)],
            pltpu.VMEM((RB, D), jnp.float32),
            pltpu.VMEM_SHARED((PAD_A, D), jnp.float32),
            [pltpu.SemaphoreType.DMA for _ in range(NB_A)],
            [pltpu.SemaphoreType.DMA for _ in range(NB_A)],
            pltpu.SemaphoreType.DMA,
            pltpu.SemaphoreType.DMA,
        ],
    )
    def k(ei_hbm, hp_hbm, out_hbm, cis, ris, bufs, stg_v, agg_sh,
          gsems, ssems, icsem, irsem):
        c = lax.axis_index("c")
        s = lax.axis_index("s")
        w = c * NSUB + s
        pltpu.sync_copy(ei_hbm.at[1, w * NSB, 0], cis[0])
        pltpu.sync_copy(ei_hbm.at[0, w * NSB, 0], ris[0])
        zero16 = jnp.zeros((16,), jnp.float32)

        def zb(i, _):
            r = i // (D // 16)
            q = i % (D // 16)
            stg_v[r, pl.ds(q * 16, 16)] = zero16
            return 0

        lax.fori_loop(0, RB * (D // 16), zb, 0)
        for t in range(STRIPE_A // RB):
            pltpu.sync_copy(stg_v, agg_sh.at[pl.ds(s * STRIPE_A + t * RB, RB)])
        plsc.subcore_barrier()

        for b in range(NB_A):
            pltpu.async_copy(
                hp_hbm.at[cis[0].at[pl.ds(b * CH_A, CH_A)]], bufs[b], gsems[b]
            )

        for sb in range(NSB):
            ccur, rcur = cis[sb % 2], ris[sb % 2]
            cnxt, rnxt = cis[(sb + 1) % 2], ris[(sb + 1) % 2]

            def body(g, _, sb=sb, ccur=ccur, rcur=rcur, cnxt=cnxt, rnxt=rnxt):
                if sb < NSB - 1:

                    @pl.when(g == 1)
                    def _load_idx():
                        pltpu.async_copy(ei_hbm.at[1, w * NSB + sb + 1, 0], cnxt, icsem)
                        pltpu.async_copy(ei_hbm.at[0, w * NSB + sb + 1, 0], rnxt, irsem)

                    @pl.when(g == GSB_A - 1)
                    def _wait_idx():
                        pltpu.make_async_copy(
                            ei_hbm.at[1, w * NSB + sb + 1, 0], cnxt, icsem
                        ).wait()
                        pltpu.make_async_copy(
                            ei_hbm.at[0, w * NSB + sb + 1, 0], rnxt, irsem
                        ).wait()

                for b in range(NB_A):
                    row = g * NB_A + b
                    pltpu.make_async_copy(
                        hp_hbm.at[ccur.at[pl.ds(row * CH_A, CH_A)]],
                        bufs[b], gsems[b],
                    ).wait()
                    pltpu.async_copy(
                        bufs[b],
                        agg_sh.at[rcur.at[pl.ds(row * CH_A, CH_A)]],
                        ssems[b], add=True,
                    )
                    pltpu.make_async_copy(
                        bufs[b],
                        agg_sh.at[rcur.at[pl.ds(row * CH_A, CH_A)]],
                        ssems[b],
                    ).wait()

                    if sb < NSB - 1:

                        @pl.when(g < GSB_A - 1)
                        def _pf_same(b=b):
                            rown = (g + 1) * NB_A + b
                            pltpu.async_copy(
                                hp_hbm.at[ccur.at[pl.ds(rown * CH_A, CH_A)]],
                                bufs[b], gsems[b],
                            )

                        @pl.when(g == GSB_A - 1)
                        def _pf_next(b=b):
                            pltpu.async_copy(
                                hp_hbm.at[cnxt.at[pl.ds(b * CH_A, CH_A)]],
                                bufs[b], gsems[b],
                            )

                    else:

                        @pl.when(g < GSB_A - 1)
                        def _pf_same(b=b):
                            rown = (g + 1) * NB_A + b
                            pltpu.async_copy(
                                hp_hbm.at[ccur.at[pl.ds(rown * CH_A, CH_A)]],
                                bufs[b], gsems[b],
                            )

                return 0

            lax.fori_loop(0, GSB_A, body, 0)

        plsc.subcore_barrier()
        pltpu.sync_copy(
            agg_sh.at[pl.ds(s * STRIPE_A, STRIPE_A)],
            out_hbm.at[c, pl.ds(s * STRIPE_A, STRIPE_A)],
        )

    return k(ei4a, hp)


_M_BLK = 1000


def _tc_hprime(x, w, deg_col):
    def body(x_ref, w_ref, d_ref, o_ref):
        dis = lax.rsqrt(d_ref[...])
        h = jnp.dot(x_ref[...], w_ref[...], preferred_element_type=jnp.float32)
        o_ref[...] = dis * h

    return pl.pallas_call(
        body,
        grid=(N // _M_BLK,),
        in_specs=[
            pl.BlockSpec((_M_BLK, D), lambda i: (i, 0)),
            pl.BlockSpec((D, D), lambda i: (0, 0)),
            pl.BlockSpec((_M_BLK, 1), lambda i: (i, 0)),
        ],
        out_specs=pl.BlockSpec((_M_BLK, D), lambda i: (i, 0)),
        out_shape=jax.ShapeDtypeStruct((N, D), jnp.float32),
    )(x, w, deg_col)


def _tc_final(aggp, hp, deg_col, bias2d):
    def body(a_ref, h_ref, d_ref, b_ref, o_ref):
        dis = lax.rsqrt(d_ref[...])
        o_ref[...] = dis * (a_ref[0] + a_ref[1] + h_ref[...]) + b_ref[...]

    return pl.pallas_call(
        body,
        grid=(N // _M_BLK,),
        in_specs=[
            pl.BlockSpec((NCORES, _M_BLK, D), lambda i: (0, i, 0)),
            pl.BlockSpec((_M_BLK, D), lambda i: (i, 0)),
            pl.BlockSpec((_M_BLK, 1), lambda i: (i, 0)),
            pl.BlockSpec((1, D), lambda i: (0, 0)),
        ],
        out_specs=pl.BlockSpec((_M_BLK, D), lambda i: (i, 0)),
        out_shape=jax.ShapeDtypeStruct((N, D), jnp.float32),
    )(aggp, hp, deg_col, bias2d)


def kernel(x, edge_index, kernel, bias):
    ei4d = edge_index.reshape(2, NW * NSB_D, 1, SBW_D)
    ei4a = edge_index.reshape(2, NW * NSB, 1, SBW_A)
    degp = _sc_degree(ei4d)
    deg_col = (degp[0, :N] + degp[1, :N] + 1.0)[:, None]
    hp = _tc_hprime(x, kernel, deg_col)
    aggp = _sc_aggregate(ei4a, hp)
    return _tc_final(aggp, hp, deg_col, bias[None, :])
